# baseline (speedup 1.0000x reference)
import functools

import jax
import jax.numpy as jnp
from jax import lax
from jax.experimental import pallas as pl
from jax.experimental.pallas import tpu as pltpu
from jax.experimental.pallas import tpu_sc as plsc

_info = plsc.get_sparse_core_info()
_NC, _NS = _info.num_cores, _info.num_subcores
_NW = _NC * _NS


@functools.lru_cache(maxsize=None)
def _make_gather(n_rows, n_table, d_model, chunk, nbuf):
    assert n_rows % (_NW * chunk) == 0
    bpw = n_rows // _NW
    n_chunks = bpw // chunk
    assert n_chunks >= nbuf and n_chunks % nbuf == 0

    mesh = plsc.VectorSubcoreMesh(core_axis_name="c", subcore_axis_name="s")

    @functools.partial(
        pl.kernel,
        out_type=jax.ShapeDtypeStruct((n_rows, d_model), jnp.float32),
        mesh=mesh,
        scratch_types=[
            pltpu.VMEM((bpw,), jnp.int32),
            pltpu.VMEM((nbuf, chunk, d_model), jnp.float32),
            pltpu.VMEM_SHARED((n_table, d_model), jnp.float32),
            [pltpu.SemaphoreType.DMA] * nbuf,
            [pltpu.SemaphoreType.DMA] * nbuf,
        ],
        compiler_params=pltpu.CompilerParams(use_tc_tiling_on_sc=False),
    )
    def gather(idx_hbm, table_hbm, out_hbm, idx_v, rows_v, tab_sh,
               gsems, ssems):
        sid = lax.axis_index("s")
        wid = sid * _NC + lax.axis_index("c")
        base = wid * bpw

        @pl.when(sid == 0)
        def _stage_table():
            pltpu.sync_copy(table_hbm, tab_sh)

        plsc.subcore_barrier()

        def src_tab(b):
            return tab_sh if b % 2 == 0 else table_hbm

        def start_gather(g, b):
            pltpu.async_copy(
                src_tab(b).at[idx_v.at[pl.ds(g * chunk, chunk)]],
                rows_v.at[b], gsems[b])

        def wait_gather(b):
            pltpu.make_async_copy(
                src_tab(b).at[idx_v.at[pl.ds(0, chunk)]],
                rows_v.at[b], gsems[b]).wait()

        def start_store(g, b):
            pltpu.async_copy(rows_v.at[b],
                             out_hbm.at[pl.ds(base + g * chunk, chunk)],
                             ssems[b])

        def wait_store(b):
            pltpu.make_async_copy(rows_v.at[b],
                                  out_hbm.at[pl.ds(base, chunk)],
                                  ssems[b]).wait()

        pltpu.sync_copy(idx_hbm.at[pl.ds(base, bpw)], idx_v)
        for b in range(nbuf - 1):
            start_gather(b, b)

        def step(gg, carry):
            for b in range(nbuf):
                g = gg * nbuf + b
                nb = (b + nbuf - 1) % nbuf

                @pl.when(g + nbuf - 1 < n_chunks)
                def _prefetch():
                    @pl.when(g >= 1)
                    def _reclaim():
                        wait_store(nb)

                    start_gather(g + nbuf - 1, nb)

                wait_gather(b)
                start_store(g, b)
            return carry

        lax.fori_loop(0, n_chunks // nbuf, step, 0)
        for b in range(nbuf):
            wait_store(b)

    return gather


def kernel(indices, pe):
    b, l = indices.shape
    d_model = pe.shape[1]
    flat = indices.reshape(-1)
    out = _make_gather(b * l, pe.shape[0], d_model, 256, 4)(flat, pe)
    return out.reshape(b, l, d_model)


# speedup vs baseline: 1.0722x; 1.0722x over previous
import functools

import jax
import jax.numpy as jnp
from jax import lax
from jax.experimental import pallas as pl
from jax.experimental.pallas import tpu as pltpu
from jax.experimental.pallas import tpu_sc as plsc

_info = plsc.get_sparse_core_info()
_NC, _NS = _info.num_cores, _info.num_subcores
_NW = _NC * _NS


@functools.lru_cache(maxsize=None)
def _make_gather(n_rows, n_table, d_model, chunk, nbuf):
    assert n_rows % (_NW * chunk) == 0
    bpw = n_rows // _NW
    n_chunks = bpw // chunk
    assert n_chunks >= nbuf and n_chunks % nbuf == 0

    mesh = plsc.VectorSubcoreMesh(core_axis_name="c", subcore_axis_name="s")

    @functools.partial(
        pl.kernel,
        out_type=jax.ShapeDtypeStruct((n_rows, d_model), jnp.float32),
        mesh=mesh,
        scratch_types=[
            pltpu.VMEM((bpw,), jnp.int32),
            pltpu.VMEM((nbuf, chunk, d_model), jnp.float32),
            pltpu.VMEM_SHARED((n_table, d_model), jnp.float32),
            [pltpu.SemaphoreType.DMA] * nbuf,
            [pltpu.SemaphoreType.DMA] * nbuf,
        ],
        compiler_params=pltpu.CompilerParams(use_tc_tiling_on_sc=False),
    )
    def gather(idx_hbm, table_hbm, out_hbm, idx_v, rows_v, tab_sh,
               gsems, ssems):
        sid = lax.axis_index("s")
        wid = sid * _NC + lax.axis_index("c")
        base = wid * bpw

        @pl.when(sid == 0)
        def _stage_table():
            pltpu.sync_copy(table_hbm, tab_sh)

        plsc.subcore_barrier()

        def start_gather(g, b):
            pltpu.async_copy(tab_sh.at[idx_v.at[pl.ds(g * chunk, chunk)]],
                             rows_v.at[b], gsems[b])

        def wait_gather(b):
            pltpu.make_async_copy(tab_sh.at[idx_v.at[pl.ds(0, chunk)]],
                                  rows_v.at[b], gsems[b]).wait()

        def start_store(g, b):
            pltpu.async_copy(rows_v.at[b],
                             out_hbm.at[pl.ds(base + g * chunk, chunk)],
                             ssems[b])

        def wait_store(b):
            pltpu.make_async_copy(rows_v.at[b],
                                  out_hbm.at[pl.ds(base, chunk)],
                                  ssems[b]).wait()

        pltpu.sync_copy(idx_hbm.at[pl.ds(base, bpw)], idx_v)
        for b in range(nbuf - 1):
            start_gather(b, b)

        def step(gg, carry):
            for b in range(nbuf):
                g = gg * nbuf + b
                nb = (b + nbuf - 1) % nbuf

                @pl.when(g + nbuf - 1 < n_chunks)
                def _prefetch():
                    @pl.when(g >= 1)
                    def _reclaim():
                        wait_store(nb)

                    start_gather(g + nbuf - 1, nb)

                wait_gather(b)
                start_store(g, b)
            return carry

        lax.fori_loop(0, n_chunks // nbuf, step, 0)
        for b in range(nbuf):
            wait_store(b)

    return gather


def kernel(indices, pe):
    b, l = indices.shape
    d_model = pe.shape[1]
    flat = indices.reshape(-1)
    out = _make_gather(b * l, pe.shape[0], d_model, 256, 4)(flat, pe)
    return out.reshape(b, l, d_model)
